# baseline (speedup 1.0000x reference)
import functools

import jax
import jax.numpy as jnp
from jax import lax
from jax.experimental import pallas as pl
from jax.experimental.pallas import tpu as pltpu
from jax.experimental.pallas import tpu_sc as plsc

N_TOKENS = 16384
HIDDEN = 2048
N_EXPERTS = 64
TOP_K = 2
SCALE = 2.5

TM = 512
NCHUNK = 4
CHUNK = N_TOKENS // NCHUNK
BPC = CHUNK // TM

NC, NS, L = 2, 16, 16
NW = NC * NS


def _mm_body(x_ref, w_ref, lo_ref, lot_ref):
    logits = lax.dot_general(
        x_ref[...], w_ref[...], (((1,), (1,)), ((), ())),
        preferred_element_type=jnp.float32,
    )
    lo_ref[...] = logits
    lot_ref[...] = logits.T


_CPARAMS = pltpu.CompilerParams(dimension_semantics=("arbitrary",))


def _make_mm(c):
    return pl.pallas_call(
        _mm_body,
        grid=(BPC,),
        in_specs=[
            pl.BlockSpec((TM, HIDDEN), lambda i, c=c: (c * BPC + i, 0)),
            pl.BlockSpec((N_EXPERTS, HIDDEN), lambda i: (0, 0)),
        ],
        out_specs=[
            pl.BlockSpec((TM, N_EXPERTS), lambda i: (i, 0)),
            pl.BlockSpec((N_EXPERTS, TM), lambda i: (0, i)),
        ],
        out_shape=[
            jax.ShapeDtypeStruct((CHUNK, N_EXPERTS), jnp.float32),
            jax.ShapeDtypeStruct((N_EXPERTS, CHUNK), jnp.float32),
        ],
        compiler_params=_CPARAMS,
    )


_MM = [_make_mm(c) for c in range(NCHUNK)]

RPW = CHUNK // NW
NBLK = RPW // L


@functools.partial(
    pl.kernel,
    out_type=[
        jax.ShapeDtypeStruct((TOP_K, CHUNK), jnp.float32),
        jax.ShapeDtypeStruct((TOP_K, CHUNK), jnp.int32),
    ],
    mesh=plsc.VectorSubcoreMesh(
        core_axis_name="c", subcore_axis_name="s",
        num_cores=NC, num_subcores=NS,
    ),
    scratch_types=[
        pltpu.VMEM((N_EXPERTS, RPW), jnp.float32),
        pltpu.VMEM((N_EXPERTS, L), jnp.float32),
        pltpu.VMEM((RPW,), jnp.float32),
        pltpu.VMEM((RPW,), jnp.float32),
        pltpu.VMEM((RPW,), jnp.int32),
        pltpu.VMEM((RPW,), jnp.int32),
    ],
)
def _sc_router(logitsT_hbm, biasb_hbm, twt_hbm, tit_hbm,
               slabT, biasb, w1s, w2s, i1s, i2s):
    wid = lax.axis_index("s") * NC + lax.axis_index("c")
    base = wid * RPW
    pltpu.sync_copy(logitsT_hbm.at[:, pl.ds(base, RPW)], slabT)
    pltpu.sync_copy(biasb_hbm, biasb)

    neg_inf = jnp.full((L,), -jnp.inf, jnp.float32)
    zero_i = jnp.zeros((L,), jnp.int32)

    def block(j, carry):
        sl = pl.ds(j * L, L)
        m = neg_inf
        for e in range(N_EXPERTS):
            m = jnp.maximum(m, slabT[e, sl])
        ssum = jnp.zeros((L,), jnp.float32)
        for e in range(N_EXPERTS):
            z = jnp.exp(slabT[e, sl] - m)
            slabT[e, sl] = z
            ssum = ssum + z
        rinv = 1.0 / ssum
        s1 = neg_inf
        s2 = neg_inf
        w1 = jnp.zeros((L,), jnp.float32)
        w2 = jnp.zeros((L,), jnp.float32)
        i1 = zero_i
        i2 = zero_i
        for e in range(N_EXPERTS):
            sc = slabT[e, sl] * rinv
            s = sc + biasb[e]
            ecol = jnp.full((L,), e, jnp.int32)
            gt1 = s > s1
            gt2 = s > s2
            s2 = jnp.where(gt1, s1, jnp.where(gt2, s, s2))
            w2 = jnp.where(gt1, w1, jnp.where(gt2, sc, w2))
            i2 = jnp.where(gt1, i1, jnp.where(gt2, ecol, i2))
            s1 = jnp.where(gt1, s, s1)
            w1 = jnp.where(gt1, sc, w1)
            i1 = jnp.where(gt1, ecol, i1)
        w1s[sl] = w1 * SCALE
        w2s[sl] = w2 * SCALE
        i1s[sl] = i1
        i2s[sl] = i2
        return carry

    lax.fori_loop(0, NBLK, block, 0)
    pltpu.sync_copy(w1s, twt_hbm.at[0, pl.ds(base, RPW)])
    pltpu.sync_copy(w2s, twt_hbm.at[1, pl.ds(base, RPW)])
    pltpu.sync_copy(i1s, tit_hbm.at[0, pl.ds(base, RPW)])
    pltpu.sync_copy(i2s, tit_hbm.at[1, pl.ds(base, RPW)])


def kernel(hidden_states, W, e_score_correction_bias):
    biasb = jnp.broadcast_to(e_score_correction_bias[:, None], (N_EXPERTS, L))
    ls, twts, tits = [], [], []
    for c in range(NCHUNK):
        l_c, logitsT = _MM[c](hidden_states, W)
        twt, tit = _sc_router(logitsT, biasb)
        ls.append(l_c)
        twts.append(twt)
        tits.append(tit)
    logits = jnp.concatenate(ls, axis=0)
    topk_weights = jnp.concatenate(twts, axis=1).T
    topk_indices = jnp.concatenate(tits, axis=1).T
    return (logits, topk_weights, topk_indices)


# speedup vs baseline: 1.2087x; 1.2087x over previous
import functools

import jax
import jax.numpy as jnp
from jax import lax
from jax.experimental import pallas as pl
from jax.experimental.pallas import tpu as pltpu
from jax.experimental.pallas import tpu_sc as plsc

N_TOKENS = 16384
HIDDEN = 2048
N_EXPERTS = 64
TOP_K = 2
SCALE = 2.5

TM = 512

NC, NS, L = 2, 16, 16
NW = NC * NS
RPW = N_TOKENS // NW
NBLK = RPW // L


def _mm_body(x_ref, w_ref, lo_ref, lot_ref):
    logits = lax.dot_general(
        x_ref[...], w_ref[...], (((1,), (1,)), ((), ())),
        preferred_element_type=jnp.float32,
    )
    lo_ref[...] = logits
    lot_ref[...] = logits.T


_matmul = pl.pallas_call(
    _mm_body,
    grid=(N_TOKENS // TM,),
    in_specs=[
        pl.BlockSpec((TM, HIDDEN), lambda i: (i, 0)),
        pl.BlockSpec((N_EXPERTS, HIDDEN), lambda i: (0, 0)),
    ],
    out_specs=[
        pl.BlockSpec((TM, N_EXPERTS), lambda i: (i, 0)),
        pl.BlockSpec((N_EXPERTS, TM), lambda i: (0, i)),
    ],
    out_shape=[
        jax.ShapeDtypeStruct((N_TOKENS, N_EXPERTS), jnp.float32),
        jax.ShapeDtypeStruct((N_EXPERTS, N_TOKENS), jnp.float32),
    ],
    compiler_params=pltpu.CompilerParams(
        dimension_semantics=("arbitrary",),
    ),
)


@functools.partial(
    pl.kernel,
    out_type=[
        jax.ShapeDtypeStruct((TOP_K, N_TOKENS), jnp.float32),
        jax.ShapeDtypeStruct((TOP_K, N_TOKENS), jnp.int32),
    ],
    mesh=plsc.VectorSubcoreMesh(
        core_axis_name="c", subcore_axis_name="s",
        num_cores=NC, num_subcores=NS,
    ),
    scratch_types=[
        pltpu.VMEM((N_EXPERTS, RPW), jnp.float32),
        pltpu.VMEM((N_EXPERTS, 128), jnp.float32),
        pltpu.VMEM((RPW,), jnp.float32),
        pltpu.VMEM((RPW,), jnp.float32),
        pltpu.VMEM((RPW,), jnp.int32),
        pltpu.VMEM((RPW,), jnp.int32),
    ],
    compiler_params=pltpu.CompilerParams(use_tc_tiling_on_sc=True),
)
def _sc_router(logitsT_hbm, biasb_hbm, twt_hbm, tit_hbm,
               slabT, biasb, w1s, w2s, i1s, i2s):
    wid = lax.axis_index("s") * NC + lax.axis_index("c")
    base = wid * RPW
    pltpu.sync_copy(logitsT_hbm.at[:, pl.ds(base, RPW)], slabT)
    pltpu.sync_copy(biasb_hbm, biasb)

    neg_inf = jnp.full((L,), -jnp.inf, jnp.float32)
    zero_i = jnp.zeros((L,), jnp.int32)

    def block(j, carry):
        sl = pl.ds(j * L, L)
        m = neg_inf
        for e in range(N_EXPERTS):
            m = jnp.maximum(m, slabT[e, sl])
        ssum = jnp.zeros((L,), jnp.float32)
        for e in range(N_EXPERTS):
            z = jnp.exp(slabT[e, sl] - m)
            slabT[e, sl] = z
            ssum = ssum + z
        rinv = 1.0 / ssum
        s1 = neg_inf
        s2 = neg_inf
        w1 = jnp.zeros((L,), jnp.float32)
        w2 = jnp.zeros((L,), jnp.float32)
        i1 = zero_i
        i2 = zero_i
        for e in range(N_EXPERTS):
            sc = slabT[e, sl] * rinv
            s = sc + biasb[e, :L]
            ecol = jnp.full((L,), e, jnp.int32)
            gt1 = s > s1
            gt2 = s > s2
            s2 = jnp.where(gt1, s1, jnp.where(gt2, s, s2))
            w2 = jnp.where(gt1, w1, jnp.where(gt2, sc, w2))
            i2 = jnp.where(gt1, i1, jnp.where(gt2, ecol, i2))
            s1 = jnp.where(gt1, s, s1)
            w1 = jnp.where(gt1, sc, w1)
            i1 = jnp.where(gt1, ecol, i1)
        w1s[sl] = w1 * SCALE
        w2s[sl] = w2 * SCALE
        i1s[sl] = i1
        i2s[sl] = i2
        return carry

    lax.fori_loop(0, NBLK, block, 0)
    pltpu.sync_copy(w1s, twt_hbm.at[0, pl.ds(base, RPW)])
    pltpu.sync_copy(w2s, twt_hbm.at[1, pl.ds(base, RPW)])
    pltpu.sync_copy(i1s, tit_hbm.at[0, pl.ds(base, RPW)])
    pltpu.sync_copy(i2s, tit_hbm.at[1, pl.ds(base, RPW)])


def kernel(hidden_states, W, e_score_correction_bias):
    logits, logitsT = _matmul(hidden_states, W)
    biasb = jnp.broadcast_to(e_score_correction_bias[:, None], (N_EXPERTS, 128))
    twt, tit = _sc_router(logitsT, biasb)
    return (logits, twt.T, tit.T)
